# baseline (speedup 1.0000x reference)
import functools

import jax
import jax.numpy as jnp
from jax import lax
from jax.experimental import pallas as pl
from jax.experimental.pallas import tpu as pltpu
from jax.experimental.pallas import tpu_sc as plsc

NUM_CORES = 2
NUM_SUBCORES = 16
NUM_WORKERS = NUM_CORES * NUM_SUBCORES
BLK = 128
LANES = 16
UNROLL = 6
ROWS_PER_TILE = 640
NPAD = NUM_SUBCORES * ROWS_PER_TILE


def _sc_aggregate(x, epk, valp, steps):
    n, d = x.shape
    nvec = d // LANES
    nz = ROWS_PER_TILE // BLK
    mesh = plsc.VectorSubcoreMesh(core_axis_name="c", subcore_axis_name="s")

    @functools.partial(
        pl.kernel,
        out_type=jax.ShapeDtypeStruct((NUM_CORES, NPAD, d), jnp.float32),
        mesh=mesh,
        scratch_types=[
            pltpu.VMEM((3, 2, BLK), jnp.int32),
            pltpu.VMEM((3, BLK), jnp.float32),
            pltpu.VMEM((2, BLK, d), jnp.float32),
            pltpu.VMEM_SHARED((NPAD, d), jnp.float32),
            pltpu.SemaphoreType.DMA((3,)),
            pltpu.SemaphoreType.DMA((3,)),
            pltpu.SemaphoreType.DMA((2,)),
            pltpu.SemaphoreType.DMA((2,)),
        ],
    )
    def body(x_hbm, epk_hbm, valp_hbm, out_hbm,
             pkbuf, vbuf, gath, acc, psem, vsem, gsem, ssem):
        c = lax.axis_index("c")
        s = lax.axis_index("s")
        wid = s * NUM_CORES + c

        def zero_body(i, carry):
            for k in range(nvec):
                sl = pl.ds(k * LANES, LANES)
                z = jnp.zeros((LANES,), jnp.float32)
                gath[1, i, sl] = z
            return carry

        lax.fori_loop(0, BLK, zero_body, 0)
        base = s * ROWS_PER_TILE
        for k in range(nz):
            pltpu.sync_copy(gath.at[1], acc.at[pl.ds(base + k * BLK, BLK)])

        pltpu.async_copy(epk_hbm.at[wid, 0], pkbuf.at[0], psem.at[0])
        pltpu.async_copy(valp_hbm.at[wid, 0], vbuf.at[0], vsem.at[0])
        pltpu.async_copy(epk_hbm.at[wid, 1], pkbuf.at[1], psem.at[1])
        pltpu.async_copy(valp_hbm.at[wid, 1], vbuf.at[1], vsem.at[1])
        plsc.subcore_barrier()
        pltpu.make_async_copy(epk_hbm.at[wid, 0], pkbuf.at[0], psem.at[0]).wait()
        pltpu.make_async_copy(valp_hbm.at[wid, 0], vbuf.at[0], vsem.at[0]).wait()
        pltpu.async_copy(gath.at[1], acc.at[pkbuf.at[0, 0]], ssem.at[1], add=True)
        pltpu.async_copy(x_hbm.at[pkbuf.at[0, 1]], gath.at[0], gsem.at[0])

        def run_phase(t, ph):
            b2 = ph % 2
            nb2 = 1 - b2
            b3 = ph % 3
            p = (ph + 1) % 3
            p2 = (ph + 2) % 3
            pltpu.make_async_copy(
                x_hbm.at[pkbuf.at[b3, 1]], gath.at[b2], gsem.at[b2]).wait()

            def scale_group(g, c2):
                vblock = vbuf[b3, pl.ds(g * LANES, LANES)]
                ebase = g * LANES
                for j in range(LANES):
                    v = vblock[j]
                    for k in range(nvec):
                        sl = pl.ds(k * LANES, LANES)
                        gath[b2, ebase + j, sl] = gath[b2, ebase + j, sl] * v
                return c2

            lax.fori_loop(0, BLK // LANES, scale_group, 0)
            pltpu.async_copy(
                gath.at[b2], acc.at[pkbuf.at[b3, 0]], ssem.at[b2], add=True)
            pltpu.make_async_copy(
                gath.at[nb2], acc.at[pkbuf.at[p2, 0]], ssem.at[nb2]).wait()
            pltpu.make_async_copy(
                epk_hbm.at[wid, t + 1], pkbuf.at[p], psem.at[p]).wait()
            pltpu.make_async_copy(
                valp_hbm.at[wid, t + 1], vbuf.at[p], vsem.at[p]).wait()
            pltpu.async_copy(
                x_hbm.at[pkbuf.at[p, 1]], gath.at[nb2], gsem.at[nb2])
            pltpu.async_copy(
                epk_hbm.at[wid, t + 2], pkbuf.at[p2], psem.at[p2])
            pltpu.async_copy(
                valp_hbm.at[wid, t + 2], vbuf.at[p2], vsem.at[p2])

        def group_body(t6, carry):
            t0 = t6 * UNROLL
            for ph in range(UNROLL):
                run_phase(t0 + ph, ph)
            return carry

        lax.fori_loop(0, steps // UNROLL, group_body, 0)

        pltpu.make_async_copy(
            x_hbm.at[pkbuf.at[0, 1]], gath.at[steps % 2], gsem.at[steps % 2]).wait()
        pp = (steps + 1) % 3
        pltpu.make_async_copy(
            epk_hbm.at[wid, steps + 1], pkbuf.at[pp], psem.at[pp]).wait()
        pltpu.make_async_copy(
            valp_hbm.at[wid, steps + 1], vbuf.at[pp], vsem.at[pp]).wait()
        lb = (steps - 1) % 2
        pltpu.make_async_copy(
            gath.at[lb], acc.at[pkbuf.at[0, 0]], ssem.at[lb]).wait()
        plsc.subcore_barrier()
        sl = pl.ds(base, ROWS_PER_TILE)
        pltpu.sync_copy(acc.at[sl], out_hbm.at[c, sl])

    return body(x, epk, valp)


def _tc_linear(x, partials, w, b2):
    n, d = x.shape
    bn = 1000

    def body(x_ref, p_ref, w_ref, b_ref, o_ref):
        xb = x_ref[...]
        nb = p_ref[0] + p_ref[1]
        w1 = w_ref[:, :d]
        w2 = w_ref[:, d:]
        acc = lax.dot_general(xb, w1, (((1,), (1,)), ((), ())),
                              preferred_element_type=jnp.float32)
        acc = acc + lax.dot_general(nb, w2, (((1,), (1,)), ((), ())),
                                    preferred_element_type=jnp.float32)
        o_ref[...] = acc + b_ref[...]

    return pl.pallas_call(
        body,
        grid=(n // bn,),
        in_specs=[
            pl.BlockSpec((bn, d), lambda i: (i, 0)),
            pl.BlockSpec((NUM_CORES, bn, d), lambda i: (0, i, 0)),
            pl.BlockSpec((d, 2 * d), lambda i: (0, 0)),
            pl.BlockSpec((1, d), lambda i: (0, 0)),
        ],
        out_specs=pl.BlockSpec((bn, d), lambda i: (i, 0)),
        out_shape=jax.ShapeDtypeStruct((n, d), jnp.float32),
    )(x, partials, w, b2)


def kernel(x, adj_indices, adj_values, W, b):
    n, d = x.shape
    e = adj_values.shape[0]
    row = adj_indices[0]
    col = adj_indices[1]

    per_worker = NUM_WORKERS * BLK
    steps = -(-e // per_worker)
    steps = -(-steps // UNROLL) * UNROLL
    alloc_steps = steps + 2
    ep = alloc_steps * per_worker
    pad = ep - e
    row = jnp.concatenate([row, jnp.zeros((pad,), row.dtype)])
    col = jnp.concatenate([col, jnp.zeros((pad,), col.dtype)])
    val = jnp.concatenate([adj_values, jnp.zeros((pad,), adj_values.dtype)])
    real = steps * per_worker
    rowr = row[:real].reshape(NUM_WORKERS, steps, BLK)
    colr = col[:real].reshape(NUM_WORKERS, steps, BLK)
    valr = val[:real].reshape(NUM_WORKERS, steps, BLK)
    zi = jnp.zeros((NUM_WORKERS, 2, BLK), jnp.int32)
    zf = jnp.zeros((NUM_WORKERS, 2, BLK), jnp.float32)
    rowf = jnp.concatenate([rowr, zi], axis=1)
    colf = jnp.concatenate([colr, zi], axis=1)
    valp = jnp.concatenate([valr, zf], axis=1)
    epk = jnp.stack([rowf, colf], axis=2)

    partials = _sc_aggregate(x, epk, valp, steps)
    return _tc_linear(x, partials, W, b.reshape(1, d))


# speedup vs baseline: 3.0894x; 3.0894x over previous
import functools

import jax
import jax.numpy as jnp
from jax import lax
from jax.experimental import pallas as pl
from jax.experimental.pallas import tpu as pltpu
from jax.experimental.pallas import tpu_sc as plsc

NUM_CORES = 2
NUM_SUBCORES = 16
NUM_WORKERS = NUM_CORES * NUM_SUBCORES
BLK = 128
LANES = 16
ROWS_PER_TILE = 640
NPAD = NUM_SUBCORES * ROWS_PER_TILE


def _sc_aggregate(x, rowp, colp, valp, steps):
    n, d = x.shape
    nvec = d // LANES
    zchunk = 128
    nz = ROWS_PER_TILE // zchunk
    mesh = plsc.VectorSubcoreMesh(core_axis_name="c", subcore_axis_name="s")

    @functools.partial(
        pl.kernel,
        out_type=jax.ShapeDtypeStruct((NUM_CORES, NPAD, d), jnp.float32),
        mesh=mesh,
        scratch_types=[
            pltpu.VMEM((steps, BLK), jnp.int32),
            pltpu.VMEM((steps, BLK), jnp.int32),
            pltpu.VMEM((steps, BLK), jnp.float32),
            pltpu.VMEM((BLK, d), jnp.float32),
            pltpu.VMEM_SHARED((NPAD, d), jnp.float32),
            pltpu.SemaphoreType.DMA,
        ],
    )
    def body(x_hbm, rowp_hbm, colp_hbm, valp_hbm, out_hbm,
             row_v, col_v, val_v, gath, acc, sem):
        c = lax.axis_index("c")
        s = lax.axis_index("s")
        wid = s * NUM_CORES + c

        pltpu.sync_copy(rowp_hbm.at[wid], row_v)
        pltpu.sync_copy(colp_hbm.at[wid], col_v)
        pltpu.sync_copy(valp_hbm.at[wid], val_v)

        def zero_body(i, carry):
            for k in range(nvec):
                gath[i, pl.ds(k * LANES, LANES)] = jnp.zeros((LANES,), jnp.float32)
            return carry

        lax.fori_loop(0, zchunk, zero_body, 0)
        base = s * ROWS_PER_TILE
        for k in range(nz):
            pltpu.sync_copy(gath, acc.at[pl.ds(base + k * zchunk, zchunk)])
        plsc.subcore_barrier()

        def step_body(t, carry):
            pltpu.async_copy(x_hbm.at[col_v.at[t]], gath, sem).wait()

            def scale_group(g, c2):
                vblock = val_v[t, pl.ds(g * LANES, LANES)]
                ebase = g * LANES
                for j in range(LANES):
                    v = vblock[j]
                    for k in range(nvec):
                        sl = pl.ds(k * LANES, LANES)
                        gath[ebase + j, sl] = gath[ebase + j, sl] * v
                return c2

            lax.fori_loop(0, BLK // LANES, scale_group, 0)
            pltpu.sync_copy(gath, acc.at[row_v.at[t]], add=True)
            return carry

        lax.fori_loop(0, steps, step_body, 0)
        plsc.subcore_barrier()
        sl = pl.ds(base, ROWS_PER_TILE)
        pltpu.sync_copy(acc.at[sl], out_hbm.at[c, sl])

    return body(x, rowp, colp, valp)


def _tc_linear(x, partials, w, b2):
    n, d = x.shape
    bn = 1000

    def body(x_ref, p_ref, w_ref, b_ref, o_ref):
        xb = x_ref[...]
        nb = p_ref[0] + p_ref[1]
        w1 = w_ref[:, :d]
        w2 = w_ref[:, d:]
        acc = lax.dot_general(xb, w1, (((1,), (1,)), ((), ())),
                              preferred_element_type=jnp.float32)
        acc = acc + lax.dot_general(nb, w2, (((1,), (1,)), ((), ())),
                                    preferred_element_type=jnp.float32)
        o_ref[...] = acc + b_ref[...]

    return pl.pallas_call(
        body,
        grid=(n // bn,),
        in_specs=[
            pl.BlockSpec((bn, d), lambda i: (i, 0)),
            pl.BlockSpec((NUM_CORES, bn, d), lambda i: (0, i, 0)),
            pl.BlockSpec((d, 2 * d), lambda i: (0, 0)),
            pl.BlockSpec((1, d), lambda i: (0, 0)),
        ],
        out_specs=pl.BlockSpec((bn, d), lambda i: (i, 0)),
        out_shape=jax.ShapeDtypeStruct((n, d), jnp.float32),
    )(x, partials, w, b2)


def kernel(x, adj_indices, adj_values, W, b):
    n, d = x.shape
    e = adj_values.shape[0]
    row = adj_indices[0]
    col = adj_indices[1]

    per_worker = NUM_WORKERS * BLK
    steps = -(-e // per_worker)
    ep = steps * per_worker
    pad = ep - e
    if pad:
        row = jnp.concatenate([row, jnp.zeros((pad,), row.dtype)])
        col = jnp.concatenate([col, jnp.zeros((pad,), col.dtype)])
        val = jnp.concatenate([adj_values, jnp.zeros((pad,), adj_values.dtype)])
    else:
        val = adj_values
    rowp = row.reshape(NUM_WORKERS, steps, BLK)
    colp = col.reshape(NUM_WORKERS, steps, BLK)
    valp = val.reshape(NUM_WORKERS, steps, BLK)

    partials = _sc_aggregate(x, rowp, colp, valp, steps)
    return _tc_linear(x, partials, W, b.reshape(1, d))
